# baseline (speedup 1.0000x reference)
import functools

import jax
import jax.numpy as jnp
from jax import lax
from jax.experimental import pallas as pl
from jax.experimental.pallas import tpu as pltpu
from jax.experimental.pallas import tpu_sc as plsc

_LMAP = (0, 1, 1, 1, 2, 2, 2, 2, 2, 3, 3, 3, 3, 3, 3, 3)
_N_NODES = 10000

_NC = 2
_NS = 16
_NW = _NC * _NS
_L = 16
_CT = 48
_NPASS = 7
_NPAD = _NPASS * _NW * _CT
_RCHUNK = 1280
_BATCH = 16


def _issue_gathers(nf_hbm, ea_hbm, tw_hbm, idbuf, slot,
                   nf_bufs, tw_bufs, ea_bufs, sems):
    idx_v = idbuf[pl.ds(0, _L)]
    pltpu.async_copy(nf_hbm.at[idx_v], nf_bufs.at[slot], sems[0])
    pltpu.async_copy(tw_hbm.at[idx_v], tw_bufs.at[slot], sems[1])
    pltpu.async_copy(ea_hbm.at[idx_v], ea_bufs.at[slot], sems[2])


def _wait_gathers(nf_hbm, ea_hbm, tw_hbm, slot,
                  nf_bufs, tw_bufs, ea_bufs, sems):
    pltpu.make_async_copy(nf_hbm.at[pl.ds(0, _BATCH)],
                          nf_bufs.at[slot], sems[0]).wait()
    pltpu.make_async_copy(tw_hbm.at[pl.ds(0, _BATCH)],
                          tw_bufs.at[slot], sems[1]).wait()
    pltpu.make_async_copy(ea_hbm.at[pl.ds(0, _BATCH)],
                          ea_bufs.at[slot], sems[2]).wait()


def _accumulate(pslot, locsave, nf_bufs, tw_bufs, ea_bufs, acc, n_edges):
    loc_v = locsave[pslot, pl.ds(0, _L)]

    def edge_fn(i, _):
        iv = jnp.full((_L,), 0, jnp.int32) + i
        loc = jnp.max(jnp.take_along_axis(loc_v, iv, 0,
                                          mode="promise_in_bounds"))
        ea_v = ea_bufs[pslot, i, pl.ds(0, _L)]
        pe = []
        for l in range(4):
            pe.append([tw_bufs[pslot, i, l, pl.ds(j * _L, _L)]
                       * nf_bufs[pslot, i, pl.ds(j * _L, _L)]
                       for j in range(8)])
        for lm in range(16):
            bv = jnp.take_along_axis(ea_v, jnp.full((_L,), lm, jnp.int32), 0,
                                     mode="promise_in_bounds")
            rows = pe[_LMAP[lm]]
            for j in range(8):
                plsc.addupdate(acc.at[loc, pl.ds(lm * 128 + j * _L, _L)],
                               bv * rows[j])
        return 0

    lax.fori_loop(0, n_edges, edge_fn, 0)


def _sc_body(nf_hbm, ea_hbm, tw_hbm, recv_hbm, out_hbm,
             idbuf, locbuf, rbuf, locsave, nf_bufs, tw_bufs, ea_bufs, acc,
             sa1, sa2, sa3, sb1, sb2, sb3, s4, *, n_edges):
    sems_a = (sa1, sa2, sa3)
    sems_b = (sb1, sb2, sb3)
    c = lax.axis_index("c")
    s = lax.axis_index("s")
    w = s * _NC + c
    nchunks = n_edges // _RCHUNK
    nvec = _RCHUNK // _L

    def pass_body(p, _):
        base = p * (_NW * _CT) + w * _CT

        def zr(i, _):
            for j in range(128):
                acc[i, pl.ds(j * _L, _L)] = jnp.zeros((_L,), jnp.float32)
            return 0
        lax.fori_loop(0, _CT, zr, 0)

        pltpu.async_copy(recv_hbm.at[pl.ds(0, _RCHUNK)], rbuf.at[0], s4)

        def chunk_body(ch, cnt):
            cur = lax.rem(ch, 2)
            pltpu.make_async_copy(recv_hbm.at[pl.ds(0, _RCHUNK)],
                                  rbuf.at[cur], s4).wait()
            nxt_off = jnp.minimum(ch + 1, nchunks - 1) * _RCHUNK
            pltpu.async_copy(recv_hbm.at[pl.ds(nxt_off, _RCHUNK)],
                             rbuf.at[lax.rem(ch + 1, 2)], s4)

            def vec_body(i, carry):
                cnt, b = carry
                rr = rbuf[cur, pl.ds(i * _L, _L)]
                rr = jnp.minimum(rr, _N_NODES - 1)
                loc = rr - base
                m = (loc >= 0) & (loc < _CT)
                ids = lax.iota(jnp.int32, _L) + (ch * _RCHUNK + i * _L)
                plsc.store_compressed(idbuf.at[pl.ds(cnt, _L)], ids, mask=m)
                plsc.store_compressed(locbuf.at[pl.ds(cnt, _L)], loc, mask=m)
                cnt = cnt + jnp.sum(m.astype(jnp.int32))

                @pl.when(cnt >= _BATCH)
                def _drain():
                    slot = lax.rem(b, 2)
                    locsave[slot, pl.ds(0, _L)] = locbuf[pl.ds(0, _L)]

                    @pl.when(slot == 0)
                    def _():
                        _issue_gathers(nf_hbm, ea_hbm, tw_hbm, idbuf, slot,
                                       nf_bufs, tw_bufs, ea_bufs, sems_a)

                    @pl.when(slot == 1)
                    def _():
                        _issue_gathers(nf_hbm, ea_hbm, tw_hbm, idbuf, slot,
                                       nf_bufs, tw_bufs, ea_bufs, sems_b)

                    idbuf[pl.ds(0, _L)] = idbuf[pl.ds(_BATCH, _L)]
                    locbuf[pl.ds(0, _L)] = locbuf[pl.ds(_BATCH, _L)]

                    @pl.when(b > 0)
                    def _():
                        pslot = lax.rem(b + 1, 2)

                        @pl.when(pslot == 0)
                        def _():
                            _wait_gathers(nf_hbm, ea_hbm, tw_hbm, pslot,
                                          nf_bufs, tw_bufs, ea_bufs, sems_a)

                        @pl.when(pslot == 1)
                        def _():
                            _wait_gathers(nf_hbm, ea_hbm, tw_hbm, pslot,
                                          nf_bufs, tw_bufs, ea_bufs, sems_b)

                        _accumulate(pslot, locsave, nf_bufs, tw_bufs,
                                    ea_bufs, acc, _BATCH)

                drained = cnt >= _BATCH
                return (jnp.where(drained, cnt - _BATCH, cnt),
                        jnp.where(drained, b + 1, b))

            return lax.fori_loop(0, nvec, vec_body, cnt)

        cnt, b = lax.fori_loop(0, nchunks, chunk_body,
                               (jnp.int32(0), jnp.int32(0)))
        pltpu.make_async_copy(recv_hbm.at[pl.ds(0, _RCHUNK)],
                              rbuf.at[lax.rem(jnp.int32(nchunks), 2)],
                              s4).wait()

        @pl.when(b > 0)
        def _():
            pslot = lax.rem(b + 1, 2)

            @pl.when(pslot == 0)
            def _():
                _wait_gathers(nf_hbm, ea_hbm, tw_hbm, pslot,
                              nf_bufs, tw_bufs, ea_bufs, sems_a)

            @pl.when(pslot == 1)
            def _():
                _wait_gathers(nf_hbm, ea_hbm, tw_hbm, pslot,
                              nf_bufs, tw_bufs, ea_bufs, sems_b)

            _accumulate(pslot, locsave, nf_bufs, tw_bufs, ea_bufs, acc,
                        _BATCH)

        @pl.when(cnt > 0)
        def _():
            slot = lax.rem(b, 2)
            idbuf[pl.ds(cnt, _L)] = jnp.zeros((_L,), jnp.int32)
            locsave[slot, pl.ds(0, _L)] = locbuf[pl.ds(0, _L)]

            @pl.when(slot == 0)
            def _():
                _issue_gathers(nf_hbm, ea_hbm, tw_hbm, idbuf, slot,
                               nf_bufs, tw_bufs, ea_bufs, sems_a)
                _wait_gathers(nf_hbm, ea_hbm, tw_hbm, slot,
                              nf_bufs, tw_bufs, ea_bufs, sems_a)

            @pl.when(slot == 1)
            def _():
                _issue_gathers(nf_hbm, ea_hbm, tw_hbm, idbuf, slot,
                               nf_bufs, tw_bufs, ea_bufs, sems_b)
                _wait_gathers(nf_hbm, ea_hbm, tw_hbm, slot,
                              nf_bufs, tw_bufs, ea_bufs, sems_b)

            _accumulate(slot, locsave, nf_bufs, tw_bufs, ea_bufs, acc, cnt)

        plsc.subcore_barrier()
        pltpu.sync_copy(acc, out_hbm.at[pl.ds(base, _CT)])
        return 0

    lax.fori_loop(0, _NPASS, pass_body, 0)


def kernel(node_feats, edge_attrs, tp_weights, receiver_list, nnodes):
    E, F = node_feats.shape
    n_lm = edge_attrs.shape[1]
    recv = receiver_list.astype(jnp.int32)
    ea_pad = jnp.pad(edge_attrs, ((0, 0), (0, 128 - n_lm)))

    mesh = plsc.VectorSubcoreMesh(core_axis_name="c", subcore_axis_name="s")
    body = functools.partial(_sc_body, n_edges=E)
    out = pl.kernel(
        body,
        out_type=jax.ShapeDtypeStruct((_NPAD, n_lm * F), jnp.float32),
        mesh=mesh,
        compiler_params=pltpu.CompilerParams(needs_layout_passes=False),
        scratch_types=[
            pltpu.VMEM((2 * _BATCH,), jnp.int32),
            pltpu.VMEM((2 * _BATCH,), jnp.int32),
            pltpu.VMEM((2, _RCHUNK), jnp.int32),
            pltpu.VMEM((2, _L), jnp.int32),
            pltpu.VMEM((2, _BATCH, F), jnp.float32),
            pltpu.VMEM((2, _BATCH, 4, F), jnp.float32),
            pltpu.VMEM((2, _BATCH, 128), jnp.float32),
            pltpu.VMEM((_CT, n_lm * F), jnp.float32),
            pltpu.SemaphoreType.DMA,
            pltpu.SemaphoreType.DMA,
            pltpu.SemaphoreType.DMA,
            pltpu.SemaphoreType.DMA,
            pltpu.SemaphoreType.DMA,
            pltpu.SemaphoreType.DMA,
            pltpu.SemaphoreType.DMA,
        ],
    )(node_feats, ea_pad, tp_weights, recv)
    return out[:_N_NODES].reshape(_N_NODES, n_lm, F)


# speedup vs baseline: 1.4077x; 1.4077x over previous
import functools

import jax
import jax.numpy as jnp
from jax import lax
from jax.experimental import pallas as pl
from jax.experimental.pallas import tpu as pltpu
from jax.experimental.pallas import tpu_sc as plsc

_LMAP = (0, 1, 1, 1, 2, 2, 2, 2, 2, 3, 3, 3, 3, 3, 3, 3)
_N_NODES = 10000

_NC = 2
_NS = 16
_NW = _NC * _NS
_L = 16
_CT = 48
_NPASS = 7
_NPAD = _NPASS * _NW * _CT
_RCHUNK = 1280
_BATCH = 32
_RING = 64


def _process_batch(nf_hbm, ea_hbm, tw_hbm, idbuf, locbuf,
                   nf_buf, tw_buf, ea_buf, acc, s1, s2, s3, n_edges):
    g1 = pltpu.async_copy(nf_hbm.at[idbuf.at[pl.ds(0, _BATCH)]], nf_buf, s1)
    g2 = pltpu.async_copy(tw_hbm.at[idbuf.at[pl.ds(0, _BATCH)]], tw_buf, s2)
    g3 = pltpu.async_copy(ea_hbm.at[idbuf.at[pl.ds(0, _BATCH)]], ea_buf, s3)
    g1.wait()
    g2.wait()
    g3.wait()

    def make_edge_fn(loc_v, di):
        def edge_fn(i, _):
            iv = jnp.full((_L,), 0, jnp.int32) + (i - di)
            loc = jnp.max(jnp.take_along_axis(loc_v, iv, 0,
                                              mode="promise_in_bounds"))
            ea_v = ea_buf[i, pl.ds(0, _L)]
            pe = []
            for l in range(4):
                pe.append([tw_buf[i, l, pl.ds(j * _L, _L)]
                           * nf_buf[i, pl.ds(j * _L, _L)]
                           for j in range(8)])
            for lm in range(16):
                bv = jnp.take_along_axis(ea_v,
                                         jnp.full((_L,), lm, jnp.int32), 0,
                                         mode="promise_in_bounds")
                rows = pe[_LMAP[lm]]
                for j in range(8):
                    plsc.addupdate(acc.at[loc, pl.ds(lm * 128 + j * _L, _L)],
                                   bv * rows[j])
            return 0
        return edge_fn

    loc_v0 = locbuf[pl.ds(0, _L)]
    loc_v1 = locbuf[pl.ds(_L, _L)]
    lax.fori_loop(0, jnp.minimum(n_edges, _L), make_edge_fn(loc_v0, 0), 0)
    lax.fori_loop(_L, jnp.maximum(n_edges, _L), make_edge_fn(loc_v1, _L), 0)


def _sc_body(nf_hbm, ea_hbm, tw_hbm, recv_hbm, out_hbm,
             idbuf, locbuf, rbuf, nf_buf, tw_buf, ea_buf, acc,
             s1, s2, s3, s4, *, n_edges):
    c = lax.axis_index("c")
    s = lax.axis_index("s")
    w = s * _NC + c
    nchunks = n_edges // _RCHUNK
    nvec = _RCHUNK // _L

    def pass_body(p, _):
        base = p * (_NW * _CT) + w * _CT

        def zr(i, _):
            for j in range(128):
                acc[i, pl.ds(j * _L, _L)] = jnp.zeros((_L,), jnp.float32)
            return 0
        lax.fori_loop(0, _CT, zr, 0)

        pltpu.async_copy(recv_hbm.at[pl.ds(0, _RCHUNK)], rbuf.at[0], s4)

        def chunk_body(ch, cnt):
            cur = lax.rem(ch, 2)
            pltpu.make_async_copy(recv_hbm.at[pl.ds(0, _RCHUNK)],
                                  rbuf.at[cur], s4).wait()
            nxt_off = jnp.minimum(ch + 1, nchunks - 1) * _RCHUNK
            pltpu.async_copy(recv_hbm.at[pl.ds(nxt_off, _RCHUNK)],
                             rbuf.at[lax.rem(ch + 1, 2)], s4)

            def vec_body(i, cnt):
                rr = rbuf[cur, pl.ds(i * _L, _L)]
                rr = jnp.minimum(rr, _N_NODES - 1)
                loc = rr - base
                m = (loc >= 0) & (loc < _CT)
                ids = lax.iota(jnp.int32, _L) + (ch * _RCHUNK + i * _L)
                plsc.store_compressed(idbuf.at[pl.ds(cnt, _L)], ids, mask=m)
                plsc.store_compressed(locbuf.at[pl.ds(cnt, _L)], loc, mask=m)
                cnt = cnt + jnp.sum(m.astype(jnp.int32))

                @pl.when(cnt >= _BATCH)
                def _drain():
                    _process_batch(nf_hbm, ea_hbm, tw_hbm, idbuf, locbuf,
                                   nf_buf, tw_buf, ea_buf, acc, s1, s2, s3,
                                   _BATCH)
                    idbuf[pl.ds(0, _L)] = idbuf[pl.ds(_BATCH, _L)]
                    locbuf[pl.ds(0, _L)] = locbuf[pl.ds(_BATCH, _L)]

                return jnp.where(cnt >= _BATCH, cnt - _BATCH, cnt)

            return lax.fori_loop(0, nvec, vec_body, cnt)

        cnt = lax.fori_loop(0, nchunks, chunk_body, jnp.int32(0))
        pltpu.make_async_copy(recv_hbm.at[pl.ds(0, _RCHUNK)],
                              rbuf.at[lax.rem(jnp.int32(nchunks), 2)],
                              s4).wait()

        idbuf[pl.ds(cnt, _L)] = jnp.zeros((_L,), jnp.int32)
        idbuf[pl.ds(cnt + _L, _L)] = jnp.zeros((_L,), jnp.int32)
        _process_batch(nf_hbm, ea_hbm, tw_hbm, idbuf, locbuf,
                       nf_buf, tw_buf, ea_buf, acc, s1, s2, s3, cnt)

        plsc.subcore_barrier()
        pltpu.sync_copy(acc, out_hbm.at[pl.ds(base, _CT)])
        return 0

    lax.fori_loop(0, _NPASS, pass_body, 0)


def kernel(node_feats, edge_attrs, tp_weights, receiver_list, nnodes):
    E, F = node_feats.shape
    n_lm = edge_attrs.shape[1]
    recv = receiver_list.astype(jnp.int32)
    ea_pad = jnp.pad(edge_attrs, ((0, 0), (0, 128 - n_lm)))

    mesh = plsc.VectorSubcoreMesh(core_axis_name="c", subcore_axis_name="s")
    body = functools.partial(_sc_body, n_edges=E)
    out = pl.kernel(
        body,
        out_type=jax.ShapeDtypeStruct((_NPAD, n_lm * F), jnp.float32),
        mesh=mesh,
        compiler_params=pltpu.CompilerParams(needs_layout_passes=False),
        scratch_types=[
            pltpu.VMEM((_RING,), jnp.int32),
            pltpu.VMEM((_RING,), jnp.int32),
            pltpu.VMEM((2, _RCHUNK), jnp.int32),
            pltpu.VMEM((_BATCH, F), jnp.float32),
            pltpu.VMEM((_BATCH, 4, F), jnp.float32),
            pltpu.VMEM((_BATCH, 128), jnp.float32),
            pltpu.VMEM((_CT, n_lm * F), jnp.float32),
            pltpu.SemaphoreType.DMA,
            pltpu.SemaphoreType.DMA,
            pltpu.SemaphoreType.DMA,
            pltpu.SemaphoreType.DMA,
        ],
    )(node_feats, ea_pad, tp_weights, recv)
    return out[:_N_NODES].reshape(_N_NODES, n_lm, F)


# speedup vs baseline: 1.7962x; 1.2759x over previous
import functools

import jax
import jax.numpy as jnp
from jax import lax
from jax.experimental import pallas as pl
from jax.experimental.pallas import tpu as pltpu
from jax.experimental.pallas import tpu_sc as plsc

_LMAP = (0, 1, 1, 1, 2, 2, 2, 2, 2, 3, 3, 3, 3, 3, 3, 3)
_N_NODES = 10000

_NC = 2
_NS = 16
_NW = _NC * _NS
_L = 16
_CT = 48
_NPASS = 7
_NPAD = _NPASS * _NW * _CT
_RCHUNK = 1280
_BATCH = 32
_RING = 64


def _process_batch(nf_hbm, ea_hbm, tw_hbm, idbuf, locbuf,
                   nf_buf, tw_buf, ea_buf, acc, s1, s2, s3, n_edges):
    g1 = pltpu.async_copy(nf_hbm.at[idbuf.at[pl.ds(0, _BATCH)]], nf_buf, s1)
    g2 = pltpu.async_copy(tw_hbm.at[idbuf.at[pl.ds(0, _BATCH)]], tw_buf, s2)
    g3 = pltpu.async_copy(ea_hbm.at[idbuf.at[pl.ds(0, _BATCH)]], ea_buf, s3)
    g1.wait()
    g2.wait()
    g3.wait()

    def make_edge_fn(loc_v, di):
        def edge_fn(i, _):
            iv = jnp.full((_L,), 0, jnp.int32) + (i - di)
            loc = jnp.max(jnp.take_along_axis(loc_v, iv, 0,
                                              mode="promise_in_bounds"))
            ea_v = ea_buf[i, pl.ds(0, _L)]
            pe = []
            for l in range(4):
                pe.append([tw_buf[i, l, pl.ds(j * _L, _L)]
                           * nf_buf[i, pl.ds(j * _L, _L)]
                           for j in range(8)])
            for lm in range(16):
                bv = jnp.take_along_axis(ea_v,
                                         jnp.full((_L,), lm, jnp.int32), 0,
                                         mode="promise_in_bounds")
                rows = pe[_LMAP[lm]]
                for j in range(8):
                    plsc.addupdate(acc.at[loc, pl.ds(lm * 128 + j * _L, _L)],
                                   bv * rows[j])
            return 0
        return edge_fn

    loc_v0 = locbuf[pl.ds(0, _L)]
    loc_v1 = locbuf[pl.ds(_L, _L)]
    lax.fori_loop(0, jnp.minimum(n_edges, _L), make_edge_fn(loc_v0, 0), 0)
    lax.fori_loop(_L, jnp.maximum(n_edges, _L), make_edge_fn(loc_v1, _L), 0)


def _sc_body(nf_hbm, ea_hbm, tw_hbm, recv_hbm, out_hbm,
             idbuf, locbuf, rbuf, nf_buf, tw_buf, ea_buf, acc,
             s1, s2, s3, s4, *, n_edges):
    c = lax.axis_index("c")
    s = lax.axis_index("s")
    w = s * _NC + c
    nchunks = n_edges // _RCHUNK
    nvec = _RCHUNK // _L

    def pass_body(p, _):
        base = p * (_NW * _CT) + w * _CT

        def zr(i, _):
            for j in range(128):
                acc[i, pl.ds(j * _L, _L)] = jnp.zeros((_L,), jnp.float32)
            return 0
        lax.fori_loop(0, _CT, zr, 0)

        pltpu.async_copy(recv_hbm.at[pl.ds(0, _RCHUNK)], rbuf.at[0], s4)

        def chunk_body(ch, cnt):
            cur = lax.rem(ch, 2)
            pltpu.make_async_copy(recv_hbm.at[pl.ds(0, _RCHUNK)],
                                  rbuf.at[cur], s4).wait()
            nxt_off = jnp.minimum(ch + 1, nchunks - 1) * _RCHUNK
            pltpu.async_copy(recv_hbm.at[pl.ds(nxt_off, _RCHUNK)],
                             rbuf.at[lax.rem(ch + 1, 2)], s4)

            def vec_body(i, cnt):
                for u in range(2):
                    rr = rbuf[cur, pl.ds((2 * i + u) * _L, _L)]
                    rr = jnp.minimum(rr, _N_NODES - 1)
                    loc = rr - base
                    m = (loc >= 0) & (loc < _CT)
                    ids = lax.iota(jnp.int32, _L) + (
                        ch * _RCHUNK + (2 * i + u) * _L)
                    plsc.store_compressed(idbuf.at[pl.ds(cnt, _L)], ids,
                                          mask=m)
                    plsc.store_compressed(locbuf.at[pl.ds(cnt, _L)], loc,
                                          mask=m)
                    cnt = cnt + jnp.sum(m.astype(jnp.int32))

                @pl.when(cnt >= _BATCH)
                def _drain():
                    _process_batch(nf_hbm, ea_hbm, tw_hbm, idbuf, locbuf,
                                   nf_buf, tw_buf, ea_buf, acc, s1, s2, s3,
                                   _BATCH)
                    idbuf[pl.ds(0, _L)] = idbuf[pl.ds(_BATCH, _L)]
                    locbuf[pl.ds(0, _L)] = locbuf[pl.ds(_BATCH, _L)]
                    idbuf[pl.ds(_L, _L)] = idbuf[pl.ds(_BATCH + _L, _L)]
                    locbuf[pl.ds(_L, _L)] = locbuf[pl.ds(_BATCH + _L, _L)]

                return jnp.where(cnt >= _BATCH, cnt - _BATCH, cnt)

            return lax.fori_loop(0, nvec // 2, vec_body, cnt)

        cnt = lax.fori_loop(0, nchunks, chunk_body, jnp.int32(0))
        pltpu.make_async_copy(recv_hbm.at[pl.ds(0, _RCHUNK)],
                              rbuf.at[lax.rem(jnp.int32(nchunks), 2)],
                              s4).wait()

        idbuf[pl.ds(cnt, _L)] = jnp.zeros((_L,), jnp.int32)
        idbuf[pl.ds(cnt + _L, _L)] = jnp.zeros((_L,), jnp.int32)
        _process_batch(nf_hbm, ea_hbm, tw_hbm, idbuf, locbuf,
                       nf_buf, tw_buf, ea_buf, acc, s1, s2, s3, cnt)

        plsc.subcore_barrier()
        pltpu.sync_copy(acc, out_hbm.at[pl.ds(base, _CT)])
        return 0

    lax.fori_loop(0, _NPASS, pass_body, 0)


def kernel(node_feats, edge_attrs, tp_weights, receiver_list, nnodes):
    E, F = node_feats.shape
    n_lm = edge_attrs.shape[1]
    recv = receiver_list.astype(jnp.int32)
    ea_pad = jnp.pad(edge_attrs, ((0, 0), (0, 128 - n_lm)))

    mesh = plsc.VectorSubcoreMesh(core_axis_name="c", subcore_axis_name="s")
    body = functools.partial(_sc_body, n_edges=E)
    out = pl.kernel(
        body,
        out_type=jax.ShapeDtypeStruct((_NPAD, n_lm * F), jnp.float32),
        mesh=mesh,
        compiler_params=pltpu.CompilerParams(needs_layout_passes=False),
        scratch_types=[
            pltpu.VMEM((_RING,), jnp.int32),
            pltpu.VMEM((_RING,), jnp.int32),
            pltpu.VMEM((2, _RCHUNK), jnp.int32),
            pltpu.VMEM((_BATCH, F), jnp.float32),
            pltpu.VMEM((_BATCH, 4, F), jnp.float32),
            pltpu.VMEM((_BATCH, 128), jnp.float32),
            pltpu.VMEM((_CT, n_lm * F), jnp.float32),
            pltpu.SemaphoreType.DMA,
            pltpu.SemaphoreType.DMA,
            pltpu.SemaphoreType.DMA,
            pltpu.SemaphoreType.DMA,
        ],
    )(node_feats, ea_pad, tp_weights, recv)
    return out[:_N_NODES].reshape(_N_NODES, n_lm, F)


# speedup vs baseline: 1.8460x; 1.0278x over previous
import functools

import jax
import jax.numpy as jnp
from jax import lax
from jax.experimental import pallas as pl
from jax.experimental.pallas import tpu as pltpu
from jax.experimental.pallas import tpu_sc as plsc

_LMAP = (0, 1, 1, 1, 2, 2, 2, 2, 2, 3, 3, 3, 3, 3, 3, 3)
_N_NODES = 10000

_NC = 2
_NS = 16
_NW = _NC * _NS
_L = 16
_CT = 48
_NPASS = 7
_NPAD = _NPASS * _NW * _CT
_RCHUNK = 1280
_BATCH = 32
_RING = 64


def _process_batch(nf_hbm, ea_hbm, tw_hbm, idbuf, locbuf,
                   nf_buf, tw_buf, ea_buf, acc, s1, s2, s3, n_edges):
    g1 = pltpu.async_copy(nf_hbm.at[idbuf.at[pl.ds(0, _BATCH)]], nf_buf, s1)
    g2 = pltpu.async_copy(tw_hbm.at[idbuf.at[pl.ds(0, _BATCH)]], tw_buf, s2)
    g3 = pltpu.async_copy(ea_hbm.at[idbuf.at[pl.ds(0, _BATCH)]], ea_buf, s3)
    g1.wait()
    g2.wait()
    g3.wait()

    def make_edge_fn(loc_v, di):
        def edge_fn(i, _):
            iv = jnp.full((_L,), 0, jnp.int32) + (i - di)
            loc = jnp.max(jnp.take_along_axis(loc_v, iv, 0,
                                              mode="promise_in_bounds"))
            ea_v = ea_buf[i, pl.ds(0, _L)]
            pe = []
            for l in range(4):
                pe.append([tw_buf[i, l, pl.ds(j * _L, _L)]
                           * nf_buf[i, pl.ds(j * _L, _L)]
                           for j in range(8)])
            for lm in range(16):
                bv = jnp.take_along_axis(ea_v,
                                         jnp.full((_L,), lm, jnp.int32), 0,
                                         mode="promise_in_bounds")
                rows = pe[_LMAP[lm]]
                for j in range(8):
                    plsc.addupdate(acc.at[loc, pl.ds(lm * 128 + j * _L, _L)],
                                   bv * rows[j])
            return 0
        return edge_fn

    loc_v0 = locbuf[pl.ds(0, _L)]
    loc_v1 = locbuf[pl.ds(_L, _L)]
    lax.fori_loop(0, jnp.minimum(n_edges, _L), make_edge_fn(loc_v0, 0), 0)
    lax.fori_loop(_L, jnp.maximum(n_edges, _L), make_edge_fn(loc_v1, _L), 0)


def _sc_body(nf_hbm, ea_hbm, tw_hbm, recv_hbm, out_hbm,
             idbuf, locbuf, rbuf, nf_buf, tw_buf, ea_buf, acc,
             s1, s2, s3, s4, *, n_edges):
    c = lax.axis_index("c")
    s = lax.axis_index("s")
    w = s * _NC + c
    nchunks = n_edges // _RCHUNK
    nvec = _RCHUNK // _L

    def pass_body(p, _):
        base = p * (_NW * _CT) + w * _CT

        def zr(i, _):
            for j in range(128):
                acc[i, pl.ds(j * _L, _L)] = jnp.zeros((_L,), jnp.float32)
            return 0
        lax.fori_loop(0, _CT, zr, 0)

        pltpu.async_copy(recv_hbm.at[pl.ds(0, _RCHUNK)], rbuf.at[0], s4)

        def chunk_body(ch, cnt):
            cur = lax.rem(ch, 2)
            pltpu.make_async_copy(recv_hbm.at[pl.ds(0, _RCHUNK)],
                                  rbuf.at[cur], s4).wait()
            nxt_off = jnp.minimum(ch + 1, nchunks - 1) * _RCHUNK
            pltpu.async_copy(recv_hbm.at[pl.ds(nxt_off, _RCHUNK)],
                             rbuf.at[lax.rem(ch + 1, 2)], s4)

            def vec_body(i, cnt):
                for h in range(2):
                    for u in range(2):
                        v = 4 * i + 2 * h + u
                        rr = rbuf[cur, pl.ds(v * _L, _L)]
                        rr = jnp.minimum(rr, _N_NODES - 1)
                        loc = rr - base
                        m = (loc >= 0) & (loc < _CT)
                        ids = lax.iota(jnp.int32, _L) + (
                            ch * _RCHUNK + v * _L)
                        plsc.store_compressed(idbuf.at[pl.ds(cnt, _L)], ids,
                                              mask=m)
                        plsc.store_compressed(locbuf.at[pl.ds(cnt, _L)], loc,
                                              mask=m)
                        cnt = cnt + jnp.sum(m.astype(jnp.int32))

                    @pl.when(cnt >= _BATCH)
                    def _drain():
                        _process_batch(nf_hbm, ea_hbm, tw_hbm, idbuf, locbuf,
                                       nf_buf, tw_buf, ea_buf, acc,
                                       s1, s2, s3, _BATCH)
                        idbuf[pl.ds(0, _L)] = idbuf[pl.ds(_BATCH, _L)]
                        locbuf[pl.ds(0, _L)] = locbuf[pl.ds(_BATCH, _L)]
                        idbuf[pl.ds(_L, _L)] = idbuf[pl.ds(_BATCH + _L, _L)]
                        locbuf[pl.ds(_L, _L)] = locbuf[pl.ds(_BATCH + _L, _L)]

                    cnt = jnp.where(cnt >= _BATCH, cnt - _BATCH, cnt)

                return cnt

            return lax.fori_loop(0, nvec // 4, vec_body, cnt)

        cnt = lax.fori_loop(0, nchunks, chunk_body, jnp.int32(0))
        pltpu.make_async_copy(recv_hbm.at[pl.ds(0, _RCHUNK)],
                              rbuf.at[lax.rem(jnp.int32(nchunks), 2)],
                              s4).wait()

        idbuf[pl.ds(cnt, _L)] = jnp.zeros((_L,), jnp.int32)
        idbuf[pl.ds(cnt + _L, _L)] = jnp.zeros((_L,), jnp.int32)
        _process_batch(nf_hbm, ea_hbm, tw_hbm, idbuf, locbuf,
                       nf_buf, tw_buf, ea_buf, acc, s1, s2, s3, cnt)

        plsc.subcore_barrier()
        pltpu.sync_copy(acc, out_hbm.at[pl.ds(base, _CT)])
        return 0

    lax.fori_loop(0, _NPASS, pass_body, 0)


def kernel(node_feats, edge_attrs, tp_weights, receiver_list, nnodes):
    E, F = node_feats.shape
    n_lm = edge_attrs.shape[1]
    recv = receiver_list.astype(jnp.int32)
    ea_pad = jnp.pad(edge_attrs, ((0, 0), (0, 128 - n_lm)))

    mesh = plsc.VectorSubcoreMesh(core_axis_name="c", subcore_axis_name="s")
    body = functools.partial(_sc_body, n_edges=E)
    out = pl.kernel(
        body,
        out_type=jax.ShapeDtypeStruct((_NPAD, n_lm * F), jnp.float32),
        mesh=mesh,
        compiler_params=pltpu.CompilerParams(needs_layout_passes=False),
        scratch_types=[
            pltpu.VMEM((_RING,), jnp.int32),
            pltpu.VMEM((_RING,), jnp.int32),
            pltpu.VMEM((2, _RCHUNK), jnp.int32),
            pltpu.VMEM((_BATCH, F), jnp.float32),
            pltpu.VMEM((_BATCH, 4, F), jnp.float32),
            pltpu.VMEM((_BATCH, 128), jnp.float32),
            pltpu.VMEM((_CT, n_lm * F), jnp.float32),
            pltpu.SemaphoreType.DMA,
            pltpu.SemaphoreType.DMA,
            pltpu.SemaphoreType.DMA,
            pltpu.SemaphoreType.DMA,
        ],
    )(node_feats, ea_pad, tp_weights, recv)
    return out[:_N_NODES].reshape(_N_NODES, n_lm, F)
